# Initial kernel scaffold; baseline (speedup 1.0000x reference)
import jax
import jax.numpy as jnp
from jax.experimental import pallas as pl


def kernel(tokens, table):
    raise NotImplementedError("write your pallas kernel here")


# speedup vs baseline: 1.6861x; 1.6861x over previous
import functools

import jax
import jax.numpy as jnp
from jax import lax
from jax.experimental import pallas as pl
from jax.experimental.pallas import tpu as pltpu
from jax.experimental.pallas import tpu_sc as plsc

_B = 16384 * 50
_D = 64
_NC = 2
_NS = 16
_NW = _NC * _NS
_BPW = _B // _NW
_CH = 128
_NCH = _BPW // _CH


def _embedding_gather(idx, table):
    mesh = plsc.VectorSubcoreMesh(core_axis_name="c", subcore_axis_name="s")

    @functools.partial(
        pl.kernel,
        mesh=mesh,
        compiler_params=pltpu.CompilerParams(use_tc_tiling_on_sc=False),
        out_type=jax.ShapeDtypeStruct((_B, _D), jnp.float32),
        scratch_types=[
            pltpu.VMEM((_BPW,), jnp.int32),
            pltpu.VMEM((_CH, _D), jnp.float32),
            pltpu.SemaphoreType.DMA,
        ],
    )
    def k(idx_hbm, table_hbm, out_hbm, idx_v, rows_v, sem):
        wid = lax.axis_index("s") * _NC + lax.axis_index("c")
        base = wid * _BPW
        pltpu.sync_copy(idx_hbm.at[pl.ds(base, _BPW)], idx_v)

        def body(c, carry):
            start = c * _CH
            pltpu.async_copy(
                table_hbm.at[idx_v.at[pl.ds(start, _CH)]], rows_v, sem
            ).wait()
            pltpu.sync_copy(rows_v, out_hbm.at[pl.ds(base + start, _CH)])
            return carry

        lax.fori_loop(0, _NCH, body, 0)

    return k(idx, table)


def kernel(tokens, table):
    idx = tokens.reshape(-1).astype(jnp.int32)
    out = _embedding_gather(idx, table)
    return out.reshape(tokens.shape + (_D,))


# speedup vs baseline: 1.8862x; 1.1187x over previous
import functools

import jax
import jax.numpy as jnp
from jax import lax
from jax.experimental import pallas as pl
from jax.experimental.pallas import tpu as pltpu
from jax.experimental.pallas import tpu_sc as plsc

_B = 16384 * 50
_D = 64
_NC = 2
_NS = 16
_NW = _NC * _NS
_BPW = _B // _NW
_CH = 128
_NCH = _BPW // _CH
_NSLOT = 8
_NROUNDS = _NCH // _NSLOT


def _embedding_gather(idx, table):
    mesh = plsc.VectorSubcoreMesh(core_axis_name="c", subcore_axis_name="s")

    @functools.partial(
        pl.kernel,
        mesh=mesh,
        compiler_params=pltpu.CompilerParams(use_tc_tiling_on_sc=False),
        out_type=jax.ShapeDtypeStruct((_B, _D), jnp.float32),
        scratch_types=[
            pltpu.VMEM((_BPW,), jnp.int32),
            pltpu.VMEM((_NSLOT, _CH, _D), jnp.float32),
            pltpu.SemaphoreType.DMA((_NSLOT,)),
            pltpu.SemaphoreType.DMA((_NSLOT,)),
        ],
    )
    def k(idx_hbm, table_hbm, out_hbm, idx_v, bufs, gsem, wsem):
        wid = lax.axis_index("s") * _NC + lax.axis_index("c")
        base = wid * _BPW
        pltpu.sync_copy(idx_hbm.at[pl.ds(base, _BPW)], idx_v)

        def gather_desc(c, b):
            return pltpu.make_async_copy(
                table_hbm.at[idx_v.at[pl.ds(c * _CH, _CH)]],
                bufs.at[b],
                gsem.at[b],
            )

        def wb_desc(c, b):
            return pltpu.make_async_copy(
                bufs.at[b],
                out_hbm.at[pl.ds(base + c * _CH, _CH)],
                wsem.at[b],
            )

        for b in range(_NSLOT):
            gather_desc(b, b).start()

        @pl.loop(0, _NROUNDS)
        def _round(g):
            c0 = g * _NSLOT
            for b in range(_NSLOT):
                gather_desc(c0 + b, b).wait()
                wb_desc(c0 + b, b).start()

            @pl.when(g < _NROUNDS - 1)
            def _prefetch():
                for b in range(_NSLOT):
                    wb_desc(c0 + b, b).wait()
                    gather_desc(c0 + _NSLOT + b, b).start()

        cl = (_NROUNDS - 1) * _NSLOT
        for b in range(_NSLOT):
            wb_desc(cl + b, b).wait()

    return k(idx, table)


def kernel(tokens, table):
    idx = tokens.reshape(-1).astype(jnp.int32)
    out = _embedding_gather(idx, table)
    return out.reshape(tokens.shape + (_D,))
